# Initial kernel scaffold; baseline (speedup 1.0000x reference)
import jax
import jax.numpy as jnp
from jax.experimental import pallas as pl


def kernel(x, edge_index, edge_attr, batch, We0, W1_0, b1_0, W2_0, b2_0, We1, W1_1, b1_1, W2_1, b2_1, Wp, bp):
    raise NotImplementedError("write your pallas kernel here")


# speedup vs baseline: 3.9139x; 3.9139x over previous
import functools

import jax
import jax.numpy as jnp
from jax import lax
from jax.experimental import pallas as pl
from jax.experimental.pallas import tpu as pltpu
from jax.experimental.pallas import tpu_sc as plsc

N = 10000
E = 320000
D = 128
DE = 16
G = 128
T = 12

_NC = 2
_NS = 16
_NW = _NC * _NS
_EPW = E // _NW
_C = 80
_NCHUNK = _EPW // _C
_RQ = 624
_TAIL = N - _NS * _RQ

_BM = 1000


def _seg_body(h_hbm, src_hbm, dst_hbm, zd_hbm, out_hbm,
              src_v, dst_v, rows_v, acc, sem):
    cid = lax.axis_index("c")
    sid = lax.axis_index("s")
    wid = cid * _NS + sid
    pltpu.sync_copy(zd_hbm, acc.at[pl.ds(sid * _RQ, _RQ)])

    @pl.when(sid == _NS - 1)
    def _():
        pltpu.sync_copy(zd_hbm.at[pl.ds(0, _TAIL)], acc.at[pl.ds(_NS * _RQ, _TAIL)])

    plsc.subcore_barrier()

    def step(i, carry):
        base = wid * _EPW + i * _C
        pltpu.sync_copy(src_hbm.at[pl.ds(base, _C)], src_v)
        pltpu.sync_copy(dst_hbm.at[pl.ds(base, _C)], dst_v)
        pltpu.async_copy(h_hbm.at[src_v], rows_v, sem).wait()
        pltpu.sync_copy(rows_v, acc.at[dst_v], add=True)
        return carry

    lax.fori_loop(0, _NCHUNK, step, 0)
    plsc.subcore_barrier()
    pltpu.sync_copy(acc.at[pl.ds(sid * _RQ, _RQ)],
                    out_hbm.at[pl.ds(cid * N + sid * _RQ, _RQ)])

    @pl.when(sid == _NS - 1)
    def _():
        pltpu.sync_copy(acc.at[pl.ds(_NS * _RQ, _TAIL)],
                        out_hbm.at[pl.ds(cid * N + _NS * _RQ, _TAIL)])


_sc_mesh = plsc.VectorSubcoreMesh(core_axis_name="c", subcore_axis_name="s")

_seg = pl.kernel(
    _seg_body,
    out_type=jax.ShapeDtypeStruct((_NC * N, D), jnp.float32),
    mesh=_sc_mesh,
    scratch_types=[
        pltpu.VMEM((_C,), jnp.int32),
        pltpu.VMEM((_C,), jnp.int32),
        pltpu.VMEM((_C, D), jnp.float32),
        pltpu.VMEM_SHARED((N, D), jnp.float32),
        pltpu.SemaphoreType.DMA,
    ],
)


_CE = 128
_NB = E // _CE
_NB_LO = _NB // _NW
_NB_XTRA = _NB % _NW


def _ea_body(dst_hbm, attr8_hbm, zd_hbm, ea_out_hbm,
             dst_v, pk_v, wide_v, ea_acc, sem):
    cid = lax.axis_index("c")
    sid = lax.axis_index("s")
    wid = cid * _NS + sid
    pltpu.sync_copy(zd_hbm, ea_acc.at[pl.ds(sid * _RQ, _RQ)])
    pltpu.sync_copy(zd_hbm.at[pl.ds(0, _CE)], wide_v)

    @pl.when(sid == _NS - 1)
    def _():
        pltpu.sync_copy(zd_hbm.at[pl.ds(0, _TAIL)], ea_acc.at[pl.ds(_NS * _RQ, _TAIL)])

    plsc.subcore_barrier()

    niter = jnp.where(wid < _NB_XTRA, _NB_LO + 1, _NB_LO)

    def step(i, carry):
        blk = wid + i * _NW
        base = blk * _CE
        pltpu.sync_copy(dst_hbm.at[pl.ds(base, _CE)], dst_v)
        pltpu.sync_copy(attr8_hbm.at[pl.ds(blk * (_CE // 8), _CE // 8)], pk_v)
        for j in range(_CE):
            wide_v[j, pl.ds(0, DE)] = pk_v[j // 8, pl.ds((j % 8) * DE, DE)]
        pltpu.sync_copy(wide_v, ea_acc.at[dst_v], add=True)
        return carry

    lax.fori_loop(0, niter, step, 0)
    plsc.subcore_barrier()
    pltpu.sync_copy(ea_acc.at[pl.ds(sid * _RQ, _RQ)],
                    ea_out_hbm.at[pl.ds(cid * N + sid * _RQ, _RQ)])

    @pl.when(sid == _NS - 1)
    def _():
        pltpu.sync_copy(ea_acc.at[pl.ds(_NS * _RQ, _TAIL)],
                        ea_out_hbm.at[pl.ds(cid * N + _NS * _RQ, _TAIL)])


_ea = pl.kernel(
    _ea_body,
    out_type=jax.ShapeDtypeStruct((_NC * N, D), jnp.float32),
    mesh=_sc_mesh,
    scratch_types=[
        pltpu.VMEM((_CE,), jnp.int32),
        pltpu.VMEM((_CE // 8, D), jnp.float32),
        pltpu.VMEM((_CE, D), jnp.float32),
        pltpu.VMEM_SHARED((N, D), jnp.float32),
        pltpu.SemaphoreType.DMA,
    ],
)


def _mlp_body(relu_out, h_ref, p_ref, ea_ref, We_ref, W1_ref, b1_ref,
              W2_ref, b2_ref, o_ref):
    z = (h_ref[...] + p_ref[0] + p_ref[1]
         + jnp.dot(ea_ref[0] + ea_ref[1], We_ref[...],
                   preferred_element_type=jnp.float32))
    hid = jnp.maximum(
        jnp.dot(z, W1_ref[...], preferred_element_type=jnp.float32)
        + b1_ref[...], 0.0)
    z2 = (jnp.dot(hid, W2_ref[...], preferred_element_type=jnp.float32)
          + b2_ref[...])
    o_ref[...] = jnp.maximum(z2, 0.0) if relu_out else z2


def _make_mlp(relu_out):
    return pl.pallas_call(
        functools.partial(_mlp_body, relu_out),
        grid=(N // _BM,),
        in_specs=[
            pl.BlockSpec((_BM, D), lambda i: (i, 0)),
            pl.BlockSpec((_NC, _BM, D), lambda i: (0, i, 0)),
            pl.BlockSpec((_NC, _BM, DE), lambda i: (0, i, 0)),
            pl.BlockSpec((DE, D), lambda i: (0, 0)),
            pl.BlockSpec((D, 2 * D), lambda i: (0, 0)),
            pl.BlockSpec((1, 2 * D), lambda i: (0, 0)),
            pl.BlockSpec((2 * D, D), lambda i: (0, 0)),
            pl.BlockSpec((1, D), lambda i: (0, 0)),
        ],
        out_specs=pl.BlockSpec((_BM, D), lambda i: (i, 0)),
        out_shape=jax.ShapeDtypeStruct((N, D), jnp.float32),
    )


_mlp_relu = _make_mlp(True)
_mlp_last = _make_mlp(False)


def _pool_body(h_ref, b_ref, Wp_ref, bp_ref, o_ref, acc_ref, cnt_ref):
    i = pl.program_id(0)

    @pl.when(i == 0)
    def _():
        acc_ref[...] = jnp.zeros_like(acc_ref)
        cnt_ref[...] = jnp.zeros_like(cnt_ref)

    gids = lax.broadcasted_iota(jnp.int32, (G, _BM), 0)
    oh = (b_ref[0] == gids).astype(jnp.float32)
    acc_ref[...] += jnp.dot(oh, h_ref[...], preferred_element_type=jnp.float32)
    cnt_ref[...] += jnp.sum(oh, axis=1, keepdims=True)

    @pl.when(i == pl.num_programs(0) - 1)
    def _():
        emb = acc_ref[...] / jnp.maximum(cnt_ref[...], 1.0)
        o_ref[...] = (jnp.dot(emb, Wp_ref[...], preferred_element_type=jnp.float32)
                      + bp_ref[...])


_pool = pl.pallas_call(
    _pool_body,
    grid=(N // _BM,),
    in_specs=[
        pl.BlockSpec((_BM, D), lambda i: (i, 0)),
        pl.BlockSpec((1, 1, _BM), lambda i: (i, 0, 0)),
        pl.BlockSpec((D, T), lambda i: (0, 0)),
        pl.BlockSpec((1, T), lambda i: (0, 0)),
    ],
    out_specs=pl.BlockSpec((G, T), lambda i: (0, 0)),
    out_shape=jax.ShapeDtypeStruct((G, T), jnp.float32),
    scratch_shapes=[
        pltpu.VMEM((G, D), jnp.float32),
        pltpu.VMEM((G, 1), jnp.float32),
    ],
)


def kernel(x, edge_index, edge_attr, batch,
           We0, W1_0, b1_0, W2_0, b2_0,
           We1, W1_1, b1_1, W2_1, b2_1,
           Wp, bp):
    src = edge_index[0]
    dst = edge_index[1]
    zd = jnp.zeros((_RQ, D), jnp.float32)
    ea = _ea(dst, edge_attr.reshape(E // 8, 8 * DE), zd).reshape(_NC, N, D)[:, :, :DE]
    p = _seg(x, src, dst, zd).reshape(_NC, N, D)
    h1 = _mlp_relu(x, p, ea, We0, W1_0, b1_0.reshape(1, -1),
                   W2_0, b2_0.reshape(1, -1))
    p2 = _seg(h1, src, dst, zd).reshape(_NC, N, D)
    h2 = _mlp_last(h1, p2, ea, We1, W1_1, b1_1.reshape(1, -1),
                   W2_1, b2_1.reshape(1, -1))
    return _pool(h2, batch.reshape(N // _BM, 1, _BM), Wp, bp.reshape(1, -1))


# speedup vs baseline: 5.9023x; 1.5080x over previous
import functools

import jax
import jax.numpy as jnp
from jax import lax
from jax.experimental import pallas as pl
from jax.experimental.pallas import tpu as pltpu
from jax.experimental.pallas import tpu_sc as plsc

N = 10000
E = 320000
D = 128
DE = 16
G = 128
T = 12

_NC = 2
_NS = 16
_NW = _NC * _NS
_EPW = E // _NW
_C = 80
_NCHUNK = _EPW // _C
_RQ = 624
_TAIL = N - _NS * _RQ

_BM = 1000


_K = 4
_NG = _NCHUNK // _K


def _seg_body(h_hbm, src_hbm, dst_hbm, zd_hbm, out_hbm, *refs):
    srcs = refs[0:_K]
    dsts = refs[_K:2 * _K]
    rows = refs[2 * _K:3 * _K]
    acc = refs[3 * _K]
    sem_i, sem_g, sem_s = refs[3 * _K + 1:3 * _K + 4]
    cid = lax.axis_index("c")
    sid = lax.axis_index("s")
    wid = cid * _NS + sid
    pltpu.sync_copy(zd_hbm, acc.at[pl.ds(sid * _RQ, _RQ)])

    @pl.when(sid == _NS - 1)
    def _():
        pltpu.sync_copy(zd_hbm.at[pl.ds(0, _TAIL)], acc.at[pl.ds(_NS * _RQ, _TAIL)])

    plsc.subcore_barrier()

    def group(g, carry):
        base0 = wid * _EPW + g * (_K * _C)
        for j in range(_K):
            pltpu.async_copy(src_hbm.at[pl.ds(base0 + j * _C, _C)], srcs[j], sem_i)
            pltpu.async_copy(dst_hbm.at[pl.ds(base0 + j * _C, _C)], dsts[j], sem_i)
        for j in range(_K):
            pltpu.make_async_copy(src_hbm.at[pl.ds(base0 + j * _C, _C)], srcs[j], sem_i).wait()
            pltpu.make_async_copy(dst_hbm.at[pl.ds(base0 + j * _C, _C)], dsts[j], sem_i).wait()
        for j in range(_K):
            pltpu.async_copy(h_hbm.at[srcs[j]], rows[j], sem_g)
        for j in range(_K):
            pltpu.make_async_copy(h_hbm.at[srcs[j]], rows[j], sem_g).wait()
        for j in range(_K):
            pltpu.async_copy(rows[j], acc.at[dsts[j]], sem_s, add=True)
        for j in range(_K):
            pltpu.make_async_copy(rows[j], acc.at[dsts[j]], sem_s).wait()
        return carry

    lax.fori_loop(0, _NG, group, 0)
    for r in range(_NG * _K, _NCHUNK):
        baser = wid * _EPW + r * _C
        pltpu.sync_copy(src_hbm.at[pl.ds(baser, _C)], srcs[0])
        pltpu.sync_copy(dst_hbm.at[pl.ds(baser, _C)], dsts[0])
        pltpu.async_copy(h_hbm.at[srcs[0]], rows[0], sem_g).wait()
        pltpu.sync_copy(rows[0], acc.at[dsts[0]], add=True)
    plsc.subcore_barrier()
    pltpu.sync_copy(acc.at[pl.ds(sid * _RQ, _RQ)],
                    out_hbm.at[pl.ds(cid * N + sid * _RQ, _RQ)])

    @pl.when(sid == _NS - 1)
    def _():
        pltpu.sync_copy(acc.at[pl.ds(_NS * _RQ, _TAIL)],
                        out_hbm.at[pl.ds(cid * N + _NS * _RQ, _TAIL)])


_sc_mesh = plsc.VectorSubcoreMesh(core_axis_name="c", subcore_axis_name="s")

_seg = pl.kernel(
    _seg_body,
    out_type=jax.ShapeDtypeStruct((_NC * N, D), jnp.float32),
    mesh=_sc_mesh,
    scratch_types=(
        [pltpu.VMEM((_C,), jnp.int32) for _ in range(2 * _K)]
        + [pltpu.VMEM((_C, D), jnp.float32) for _ in range(_K)]
        + [pltpu.VMEM_SHARED((N, D), jnp.float32)]
        + [pltpu.SemaphoreType.DMA] * 3
    ),
)


_CE = 128
_NB = E // _CE
_NB_LO = _NB // _NW
_NB_XTRA = _NB % _NW


def _ea_body(dst_hbm, attr8_hbm, zd_hbm, ea_out_hbm,
             dst_v, pk_v, wide_v, ea_acc, sem):
    cid = lax.axis_index("c")
    sid = lax.axis_index("s")
    wid = cid * _NS + sid
    pltpu.sync_copy(zd_hbm, ea_acc.at[pl.ds(sid * _RQ, _RQ)])
    pltpu.sync_copy(zd_hbm.at[pl.ds(0, _CE)], wide_v)

    @pl.when(sid == _NS - 1)
    def _():
        pltpu.sync_copy(zd_hbm.at[pl.ds(0, _TAIL)], ea_acc.at[pl.ds(_NS * _RQ, _TAIL)])

    plsc.subcore_barrier()

    niter = jnp.where(wid < _NB_XTRA, _NB_LO + 1, _NB_LO)

    def step(i, carry):
        blk = wid + i * _NW
        base = blk * _CE
        pltpu.sync_copy(dst_hbm.at[pl.ds(base, _CE)], dst_v)
        pltpu.sync_copy(attr8_hbm.at[pl.ds(blk * (_CE // 8), _CE // 8)], pk_v)
        for j in range(_CE):
            wide_v[j, pl.ds(0, DE)] = pk_v[j // 8, pl.ds((j % 8) * DE, DE)]
        pltpu.sync_copy(wide_v, ea_acc.at[dst_v], add=True)
        return carry

    lax.fori_loop(0, niter, step, 0)
    plsc.subcore_barrier()
    pltpu.sync_copy(ea_acc.at[pl.ds(sid * _RQ, _RQ)],
                    ea_out_hbm.at[pl.ds(cid * N + sid * _RQ, _RQ)])

    @pl.when(sid == _NS - 1)
    def _():
        pltpu.sync_copy(ea_acc.at[pl.ds(_NS * _RQ, _TAIL)],
                        ea_out_hbm.at[pl.ds(cid * N + _NS * _RQ, _TAIL)])


_ea = pl.kernel(
    _ea_body,
    out_type=jax.ShapeDtypeStruct((_NC * N, D), jnp.float32),
    mesh=_sc_mesh,
    scratch_types=[
        pltpu.VMEM((_CE,), jnp.int32),
        pltpu.VMEM((_CE // 8, D), jnp.float32),
        pltpu.VMEM((_CE, D), jnp.float32),
        pltpu.VMEM_SHARED((N, D), jnp.float32),
        pltpu.SemaphoreType.DMA,
    ],
)


def _mlp_body(relu_out, h_ref, p_ref, ea_ref, We_ref, W1_ref, b1_ref,
              W2_ref, b2_ref, o_ref):
    z = (h_ref[...] + p_ref[0] + p_ref[1]
         + jnp.dot(ea_ref[0] + ea_ref[1], We_ref[...],
                   preferred_element_type=jnp.float32))
    hid = jnp.maximum(
        jnp.dot(z, W1_ref[...], preferred_element_type=jnp.float32)
        + b1_ref[...], 0.0)
    z2 = (jnp.dot(hid, W2_ref[...], preferred_element_type=jnp.float32)
          + b2_ref[...])
    o_ref[...] = jnp.maximum(z2, 0.0) if relu_out else z2


def _make_mlp(relu_out):
    return pl.pallas_call(
        functools.partial(_mlp_body, relu_out),
        grid=(N // _BM,),
        in_specs=[
            pl.BlockSpec((_BM, D), lambda i: (i, 0)),
            pl.BlockSpec((_NC, _BM, D), lambda i: (0, i, 0)),
            pl.BlockSpec((_NC, _BM, DE), lambda i: (0, i, 0)),
            pl.BlockSpec((DE, D), lambda i: (0, 0)),
            pl.BlockSpec((D, 2 * D), lambda i: (0, 0)),
            pl.BlockSpec((1, 2 * D), lambda i: (0, 0)),
            pl.BlockSpec((2 * D, D), lambda i: (0, 0)),
            pl.BlockSpec((1, D), lambda i: (0, 0)),
        ],
        out_specs=pl.BlockSpec((_BM, D), lambda i: (i, 0)),
        out_shape=jax.ShapeDtypeStruct((N, D), jnp.float32),
    )


_mlp_relu = _make_mlp(True)
_mlp_last = _make_mlp(False)


def _pool_body(h_ref, b_ref, Wp_ref, bp_ref, o_ref, acc_ref, cnt_ref):
    i = pl.program_id(0)

    @pl.when(i == 0)
    def _():
        acc_ref[...] = jnp.zeros_like(acc_ref)
        cnt_ref[...] = jnp.zeros_like(cnt_ref)

    gids = lax.broadcasted_iota(jnp.int32, (G, _BM), 0)
    oh = (b_ref[0] == gids).astype(jnp.float32)
    acc_ref[...] += jnp.dot(oh, h_ref[...], preferred_element_type=jnp.float32)
    cnt_ref[...] += jnp.sum(oh, axis=1, keepdims=True)

    @pl.when(i == pl.num_programs(0) - 1)
    def _():
        emb = acc_ref[...] / jnp.maximum(cnt_ref[...], 1.0)
        o_ref[...] = (jnp.dot(emb, Wp_ref[...], preferred_element_type=jnp.float32)
                      + bp_ref[...])


_pool = pl.pallas_call(
    _pool_body,
    grid=(N // _BM,),
    in_specs=[
        pl.BlockSpec((_BM, D), lambda i: (i, 0)),
        pl.BlockSpec((1, 1, _BM), lambda i: (i, 0, 0)),
        pl.BlockSpec((D, T), lambda i: (0, 0)),
        pl.BlockSpec((1, T), lambda i: (0, 0)),
    ],
    out_specs=pl.BlockSpec((G, T), lambda i: (0, 0)),
    out_shape=jax.ShapeDtypeStruct((G, T), jnp.float32),
    scratch_shapes=[
        pltpu.VMEM((G, D), jnp.float32),
        pltpu.VMEM((G, 1), jnp.float32),
    ],
)


def kernel(x, edge_index, edge_attr, batch,
           We0, W1_0, b1_0, W2_0, b2_0,
           We1, W1_1, b1_1, W2_1, b2_1,
           Wp, bp):
    src = edge_index[0]
    dst = edge_index[1]
    zd = jnp.zeros((_RQ, D), jnp.float32)
    ea = _ea(dst, edge_attr.reshape(E // 8, 8 * DE), zd).reshape(_NC, N, D)[:, :, :DE]
    p = _seg(x, src, dst, zd).reshape(_NC, N, D)
    h1 = _mlp_relu(x, p, ea, We0, W1_0, b1_0.reshape(1, -1),
                   W2_0, b2_0.reshape(1, -1))
    p2 = _seg(h1, src, dst, zd).reshape(_NC, N, D)
    h2 = _mlp_last(h1, p2, ea, We1, W1_1, b1_1.reshape(1, -1),
                   W2_1, b2_1.reshape(1, -1))
    return _pool(h2, batch.reshape(N // _BM, 1, _BM), Wp, bp.reshape(1, -1))


# speedup vs baseline: 6.5687x; 1.1129x over previous
import functools

import jax
import jax.numpy as jnp
from jax import lax
from jax.experimental import pallas as pl
from jax.experimental.pallas import tpu as pltpu
from jax.experimental.pallas import tpu_sc as plsc

N = 10000
E = 320000
D = 128
DE = 16
G = 128
T = 12

_NC = 2
_NS = 16
_NW = _NC * _NS
_EPW = E // _NW
_C = 80
_NCHUNK = _EPW // _C
_RQ = 624
_TAIL = N - _NS * _RQ

_BM = 1000


_K = 4
_NG = _NCHUNK // _K


def _seg_body(h_hbm, src_hbm, dst_hbm, zd_hbm, out_hbm, *refs):
    srcs = refs[0:_K]
    dsts = refs[_K:2 * _K]
    rows = refs[2 * _K:3 * _K]
    acc = refs[3 * _K]
    sem_i, sem_g, sem_s = refs[3 * _K + 1:3 * _K + 4]
    cid = lax.axis_index("c")
    sid = lax.axis_index("s")
    wid = cid * _NS + sid
    pltpu.sync_copy(zd_hbm, acc.at[pl.ds(sid * _RQ, _RQ)])

    @pl.when(sid == _NS - 1)
    def _():
        pltpu.sync_copy(zd_hbm.at[pl.ds(0, _TAIL)], acc.at[pl.ds(_NS * _RQ, _TAIL)])

    plsc.subcore_barrier()

    def group(g, carry):
        base0 = wid * _EPW + g * (_K * _C)
        for j in range(_K):
            pltpu.async_copy(src_hbm.at[pl.ds(base0 + j * _C, _C)], srcs[j], sem_i)
            pltpu.async_copy(dst_hbm.at[pl.ds(base0 + j * _C, _C)], dsts[j], sem_i)
        for j in range(_K):
            pltpu.make_async_copy(src_hbm.at[pl.ds(base0 + j * _C, _C)], srcs[j], sem_i).wait()
            pltpu.make_async_copy(dst_hbm.at[pl.ds(base0 + j * _C, _C)], dsts[j], sem_i).wait()
        for j in range(_K):
            pltpu.async_copy(h_hbm.at[srcs[j]], rows[j], sem_g)
        for j in range(_K):
            pltpu.make_async_copy(h_hbm.at[srcs[j]], rows[j], sem_g).wait()
        for j in range(_K):
            pltpu.async_copy(rows[j], acc.at[dsts[j]], sem_s, add=True)
        for j in range(_K):
            pltpu.make_async_copy(rows[j], acc.at[dsts[j]], sem_s).wait()
        return carry

    lax.fori_loop(0, _NG, group, 0)
    for r in range(_NG * _K, _NCHUNK):
        baser = wid * _EPW + r * _C
        pltpu.sync_copy(src_hbm.at[pl.ds(baser, _C)], srcs[0])
        pltpu.sync_copy(dst_hbm.at[pl.ds(baser, _C)], dsts[0])
        pltpu.async_copy(h_hbm.at[srcs[0]], rows[0], sem_g).wait()
        pltpu.sync_copy(rows[0], acc.at[dsts[0]], add=True)
    plsc.subcore_barrier()
    pltpu.sync_copy(acc.at[pl.ds(sid * _RQ, _RQ)],
                    out_hbm.at[pl.ds(cid * N + sid * _RQ, _RQ)])

    @pl.when(sid == _NS - 1)
    def _():
        pltpu.sync_copy(acc.at[pl.ds(_NS * _RQ, _TAIL)],
                        out_hbm.at[pl.ds(cid * N + _NS * _RQ, _TAIL)])


_sc_mesh = plsc.VectorSubcoreMesh(core_axis_name="c", subcore_axis_name="s")

_seg = pl.kernel(
    _seg_body,
    out_type=jax.ShapeDtypeStruct((_NC * N, D), jnp.float32),
    mesh=_sc_mesh,
    scratch_types=(
        [pltpu.VMEM((_C,), jnp.int32) for _ in range(2 * _K)]
        + [pltpu.VMEM((_C, D), jnp.float32) for _ in range(_K)]
        + [pltpu.VMEM_SHARED((N, D), jnp.float32)]
        + [pltpu.SemaphoreType.DMA] * 3
    ),
)


_CE = 128
_NB = E // _CE
_NB_LO = _NB // _NW
_NB_XTRA = _NB % _NW
_NPAIR = _NB_LO // 2


def _ea_unpack(pk, wide):
    for j in range(_CE):
        wide[j, pl.ds(0, DE)] = pk[j // 8, pl.ds((j % 8) * DE, DE)]


def _ea_body(dst_hbm, attr8_hbm, zd_hbm, ea_out_hbm,
             dstA, dstB, pkA, pkB, wideA, wideB, ea_acc,
             sem_iA, sem_iB, sem_sA, sem_sB):
    cid = lax.axis_index("c")
    sid = lax.axis_index("s")
    wid = cid * _NS + sid
    blk0 = wid * _NB_LO
    pltpu.sync_copy(zd_hbm, ea_acc.at[pl.ds(sid * _RQ, _RQ)])
    pltpu.sync_copy(zd_hbm.at[pl.ds(0, _CE)], wideA)
    pltpu.sync_copy(zd_hbm.at[pl.ds(0, _CE)], wideB)

    @pl.when(sid == _NS - 1)
    def _():
        pltpu.sync_copy(zd_hbm.at[pl.ds(0, _TAIL)], ea_acc.at[pl.ds(_NS * _RQ, _TAIL)])

    plsc.subcore_barrier()

    def stage(blk, dst_v, pk_v, sem):
        pltpu.async_copy(dst_hbm.at[pl.ds(blk * _CE, _CE)], dst_v, sem)
        pltpu.async_copy(attr8_hbm.at[pl.ds(blk * (_CE // 8), _CE // 8)], pk_v, sem)

    def stage_wait(blk, dst_v, pk_v, sem):
        pltpu.make_async_copy(dst_hbm.at[pl.ds(blk * _CE, _CE)], dst_v, sem).wait()
        pltpu.make_async_copy(attr8_hbm.at[pl.ds(blk * (_CE // 8), _CE // 8)], pk_v, sem).wait()

    stage(blk0, dstA, pkA, sem_iA)
    stage_wait(blk0, dstA, pkA, sem_iA)

    def pair(j, carry):
        c0 = blk0 + 2 * j
        _ea_unpack(pkA, wideA)

        @pl.when(j > 0)
        def _():
            pltpu.make_async_copy(wideB, ea_acc.at[dstB], sem_sB).wait()

        stage(c0 + 1, dstB, pkB, sem_iB)
        pltpu.async_copy(wideA, ea_acc.at[dstA], sem_sA, add=True)
        stage_wait(c0 + 1, dstB, pkB, sem_iB)
        _ea_unpack(pkB, wideB)
        pltpu.make_async_copy(wideA, ea_acc.at[dstA], sem_sA).wait()

        @pl.when(j < _NPAIR - 1)
        def _():
            stage(c0 + 2, dstA, pkA, sem_iA)
            stage_wait(c0 + 2, dstA, pkA, sem_iA)

        pltpu.async_copy(wideB, ea_acc.at[dstB], sem_sB, add=True)
        return carry

    lax.fori_loop(0, _NPAIR, pair, 0)
    pltpu.make_async_copy(wideB, ea_acc.at[dstB], sem_sB).wait()

    @pl.when(wid < _NB_XTRA)
    def _():
        cx = _NW * _NB_LO + wid
        stage(cx, dstA, pkA, sem_iA)
        stage_wait(cx, dstA, pkA, sem_iA)
        _ea_unpack(pkA, wideA)
        pltpu.sync_copy(wideA, ea_acc.at[dstA], add=True)

    plsc.subcore_barrier()
    pltpu.sync_copy(ea_acc.at[pl.ds(sid * _RQ, _RQ)],
                    ea_out_hbm.at[pl.ds(cid * N + sid * _RQ, _RQ)])

    @pl.when(sid == _NS - 1)
    def _():
        pltpu.sync_copy(ea_acc.at[pl.ds(_NS * _RQ, _TAIL)],
                        ea_out_hbm.at[pl.ds(cid * N + _NS * _RQ, _TAIL)])


_ea = pl.kernel(
    _ea_body,
    out_type=jax.ShapeDtypeStruct((_NC * N, D), jnp.float32),
    mesh=_sc_mesh,
    scratch_types=[
        pltpu.VMEM((_CE,), jnp.int32),
        pltpu.VMEM((_CE,), jnp.int32),
        pltpu.VMEM((_CE // 8, D), jnp.float32),
        pltpu.VMEM((_CE // 8, D), jnp.float32),
        pltpu.VMEM((_CE, D), jnp.float32),
        pltpu.VMEM((_CE, D), jnp.float32),
        pltpu.VMEM_SHARED((N, D), jnp.float32),
        pltpu.SemaphoreType.DMA,
        pltpu.SemaphoreType.DMA,
        pltpu.SemaphoreType.DMA,
        pltpu.SemaphoreType.DMA,
    ],
)


def _mlp_body(relu_out, h_ref, p_ref, ea_ref, We_ref, W1_ref, b1_ref,
              W2_ref, b2_ref, o_ref):
    z = (h_ref[...] + p_ref[0] + p_ref[1]
         + jnp.dot(ea_ref[0] + ea_ref[1], We_ref[...],
                   preferred_element_type=jnp.float32))
    hid = jnp.maximum(
        jnp.dot(z, W1_ref[...], preferred_element_type=jnp.float32)
        + b1_ref[...], 0.0)
    z2 = (jnp.dot(hid, W2_ref[...], preferred_element_type=jnp.float32)
          + b2_ref[...])
    o_ref[...] = jnp.maximum(z2, 0.0) if relu_out else z2


def _make_mlp(relu_out):
    return pl.pallas_call(
        functools.partial(_mlp_body, relu_out),
        grid=(N // _BM,),
        in_specs=[
            pl.BlockSpec((_BM, D), lambda i: (i, 0)),
            pl.BlockSpec((_NC, _BM, D), lambda i: (0, i, 0)),
            pl.BlockSpec((_NC, _BM, DE), lambda i: (0, i, 0)),
            pl.BlockSpec((DE, D), lambda i: (0, 0)),
            pl.BlockSpec((D, 2 * D), lambda i: (0, 0)),
            pl.BlockSpec((1, 2 * D), lambda i: (0, 0)),
            pl.BlockSpec((2 * D, D), lambda i: (0, 0)),
            pl.BlockSpec((1, D), lambda i: (0, 0)),
        ],
        out_specs=pl.BlockSpec((_BM, D), lambda i: (i, 0)),
        out_shape=jax.ShapeDtypeStruct((N, D), jnp.float32),
    )


_mlp_relu = _make_mlp(True)
_mlp_last = _make_mlp(False)


def _pool_body(h_ref, b_ref, Wp_ref, bp_ref, o_ref, acc_ref, cnt_ref):
    i = pl.program_id(0)

    @pl.when(i == 0)
    def _():
        acc_ref[...] = jnp.zeros_like(acc_ref)
        cnt_ref[...] = jnp.zeros_like(cnt_ref)

    gids = lax.broadcasted_iota(jnp.int32, (G, _BM), 0)
    oh = (b_ref[0] == gids).astype(jnp.float32)
    acc_ref[...] += jnp.dot(oh, h_ref[...], preferred_element_type=jnp.float32)
    cnt_ref[...] += jnp.sum(oh, axis=1, keepdims=True)

    @pl.when(i == pl.num_programs(0) - 1)
    def _():
        emb = acc_ref[...] / jnp.maximum(cnt_ref[...], 1.0)
        o_ref[...] = (jnp.dot(emb, Wp_ref[...], preferred_element_type=jnp.float32)
                      + bp_ref[...])


_pool = pl.pallas_call(
    _pool_body,
    grid=(N // _BM,),
    in_specs=[
        pl.BlockSpec((_BM, D), lambda i: (i, 0)),
        pl.BlockSpec((1, 1, _BM), lambda i: (i, 0, 0)),
        pl.BlockSpec((D, T), lambda i: (0, 0)),
        pl.BlockSpec((1, T), lambda i: (0, 0)),
    ],
    out_specs=pl.BlockSpec((G, T), lambda i: (0, 0)),
    out_shape=jax.ShapeDtypeStruct((G, T), jnp.float32),
    scratch_shapes=[
        pltpu.VMEM((G, D), jnp.float32),
        pltpu.VMEM((G, 1), jnp.float32),
    ],
)


def kernel(x, edge_index, edge_attr, batch,
           We0, W1_0, b1_0, W2_0, b2_0,
           We1, W1_1, b1_1, W2_1, b2_1,
           Wp, bp):
    src = edge_index[0]
    dst = edge_index[1]
    zd = jnp.zeros((_RQ, D), jnp.float32)
    ea = _ea(dst, edge_attr.reshape(E // 8, 8 * DE), zd).reshape(_NC, N, D)[:, :, :DE]
    p = _seg(x, src, dst, zd).reshape(_NC, N, D)
    h1 = _mlp_relu(x, p, ea, We0, W1_0, b1_0.reshape(1, -1),
                   W2_0, b2_0.reshape(1, -1))
    p2 = _seg(h1, src, dst, zd).reshape(_NC, N, D)
    h2 = _mlp_last(h1, p2, ea, We1, W1_1, b1_1.reshape(1, -1),
                   W2_1, b2_1.reshape(1, -1))
    return _pool(h2, batch.reshape(N // _BM, 1, _BM), Wp, bp.reshape(1, -1))


# speedup vs baseline: 7.3321x; 1.1162x over previous
import functools

import jax
import jax.numpy as jnp
from jax import lax
from jax.experimental import pallas as pl
from jax.experimental.pallas import tpu as pltpu
from jax.experimental.pallas import tpu_sc as plsc

N = 10000
E = 320000
D = 128
DE = 16
G = 128
T = 12

_NC = 2
_NS = 16
_NW = _NC * _NS
_EPW = E // _NW
_C = 80
_NCHUNK = _EPW // _C
_RQ = 624
_TAIL = N - _NS * _RQ

_BM = 1000


_K = 4
_NG = _NCHUNK // _K


def _seg_body(h_hbm, src_hbm, dst_hbm, zd_hbm, out_hbm, *refs):
    srcs = refs[0:_K]
    dsts = refs[_K:2 * _K]
    rows = refs[2 * _K:3 * _K]
    acc = refs[3 * _K]
    sem_i, sem_g, sem_s = refs[3 * _K + 1:3 * _K + 4]
    cid = lax.axis_index("c")
    sid = lax.axis_index("s")
    wid = cid * _NS + sid
    pltpu.sync_copy(zd_hbm, acc.at[pl.ds(sid * _RQ, _RQ)])

    @pl.when(sid == _NS - 1)
    def _():
        pltpu.sync_copy(zd_hbm.at[pl.ds(0, _TAIL)], acc.at[pl.ds(_NS * _RQ, _TAIL)])

    plsc.subcore_barrier()

    def group(g, carry):
        base0 = wid * _EPW + g * (_K * _C)

        @pl.when(g > 0)
        def _():
            for j in range(_K):
                pltpu.make_async_copy(rows[j], acc.at[dsts[j]], sem_s).wait()

        for j in range(_K):
            pltpu.async_copy(src_hbm.at[pl.ds(base0 + j * _C, _C)], srcs[j], sem_i)
            pltpu.async_copy(dst_hbm.at[pl.ds(base0 + j * _C, _C)], dsts[j], sem_i)
        for j in range(_K):
            pltpu.make_async_copy(src_hbm.at[pl.ds(base0 + j * _C, _C)], srcs[j], sem_i).wait()
            pltpu.make_async_copy(dst_hbm.at[pl.ds(base0 + j * _C, _C)], dsts[j], sem_i).wait()
            pltpu.async_copy(h_hbm.at[srcs[j]], rows[j], sem_g)
        for j in range(_K):
            pltpu.make_async_copy(h_hbm.at[srcs[j]], rows[j], sem_g).wait()
            pltpu.async_copy(rows[j], acc.at[dsts[j]], sem_s, add=True)
        return carry

    lax.fori_loop(0, _NG, group, 0)
    for j in range(_K):
        pltpu.make_async_copy(rows[j], acc.at[dsts[j]], sem_s).wait()
    for r in range(_NG * _K, _NCHUNK):
        baser = wid * _EPW + r * _C
        pltpu.sync_copy(src_hbm.at[pl.ds(baser, _C)], srcs[0])
        pltpu.sync_copy(dst_hbm.at[pl.ds(baser, _C)], dsts[0])
        pltpu.async_copy(h_hbm.at[srcs[0]], rows[0], sem_g).wait()
        pltpu.sync_copy(rows[0], acc.at[dsts[0]], add=True)
    plsc.subcore_barrier()
    pltpu.sync_copy(acc.at[pl.ds(sid * _RQ, _RQ)],
                    out_hbm.at[pl.ds(cid * N + sid * _RQ, _RQ)])

    @pl.when(sid == _NS - 1)
    def _():
        pltpu.sync_copy(acc.at[pl.ds(_NS * _RQ, _TAIL)],
                        out_hbm.at[pl.ds(cid * N + _NS * _RQ, _TAIL)])


_sc_mesh = plsc.VectorSubcoreMesh(core_axis_name="c", subcore_axis_name="s")

_seg = pl.kernel(
    _seg_body,
    out_type=jax.ShapeDtypeStruct((_NC * N, D), jnp.float32),
    mesh=_sc_mesh,
    scratch_types=(
        [pltpu.VMEM((_C,), jnp.int32) for _ in range(2 * _K)]
        + [pltpu.VMEM((_C, D), jnp.float32) for _ in range(_K)]
        + [pltpu.VMEM_SHARED((N, D), jnp.float32)]
        + [pltpu.SemaphoreType.DMA] * 3
    ),
)


_CE = 128
_NB = E // _CE
_NB_LO = _NB // _NW
_NB_XTRA = _NB % _NW
_NPAIR = _NB_LO // 2


def _ea_unpack(pk, wide):
    for j in range(_CE):
        wide[j, pl.ds(0, DE)] = pk[j // 8, pl.ds((j % 8) * DE, DE)]


def _ea_body(dst_hbm, attr8_hbm, zd_hbm, ea_out_hbm,
             dstA, dstB, pkA, pkB, wideA, wideB, ea_acc,
             sem_iA, sem_iB, sem_sA, sem_sB):
    cid = lax.axis_index("c")
    sid = lax.axis_index("s")
    wid = cid * _NS + sid
    blk0 = wid * _NB_LO
    pltpu.sync_copy(zd_hbm, ea_acc.at[pl.ds(sid * _RQ, _RQ)])
    pltpu.sync_copy(zd_hbm.at[pl.ds(0, _CE)], wideA)
    pltpu.sync_copy(zd_hbm.at[pl.ds(0, _CE)], wideB)

    @pl.when(sid == _NS - 1)
    def _():
        pltpu.sync_copy(zd_hbm.at[pl.ds(0, _TAIL)], ea_acc.at[pl.ds(_NS * _RQ, _TAIL)])

    plsc.subcore_barrier()

    def stage(blk, dst_v, pk_v, sem):
        pltpu.async_copy(dst_hbm.at[pl.ds(blk * _CE, _CE)], dst_v, sem)
        pltpu.async_copy(attr8_hbm.at[pl.ds(blk * (_CE // 8), _CE // 8)], pk_v, sem)

    def stage_wait(blk, dst_v, pk_v, sem):
        pltpu.make_async_copy(dst_hbm.at[pl.ds(blk * _CE, _CE)], dst_v, sem).wait()
        pltpu.make_async_copy(attr8_hbm.at[pl.ds(blk * (_CE // 8), _CE // 8)], pk_v, sem).wait()

    stage(blk0, dstA, pkA, sem_iA)
    stage_wait(blk0, dstA, pkA, sem_iA)

    def pair(j, carry):
        c0 = blk0 + 2 * j
        _ea_unpack(pkA, wideA)

        @pl.when(j > 0)
        def _():
            pltpu.make_async_copy(wideB, ea_acc.at[dstB], sem_sB).wait()

        stage(c0 + 1, dstB, pkB, sem_iB)
        pltpu.async_copy(wideA, ea_acc.at[dstA], sem_sA, add=True)
        stage_wait(c0 + 1, dstB, pkB, sem_iB)
        _ea_unpack(pkB, wideB)
        pltpu.make_async_copy(wideA, ea_acc.at[dstA], sem_sA).wait()

        @pl.when(j < _NPAIR - 1)
        def _():
            stage(c0 + 2, dstA, pkA, sem_iA)
            stage_wait(c0 + 2, dstA, pkA, sem_iA)

        pltpu.async_copy(wideB, ea_acc.at[dstB], sem_sB, add=True)
        return carry

    lax.fori_loop(0, _NPAIR, pair, 0)
    pltpu.make_async_copy(wideB, ea_acc.at[dstB], sem_sB).wait()

    @pl.when(wid < _NB_XTRA)
    def _():
        cx = _NW * _NB_LO + wid
        stage(cx, dstA, pkA, sem_iA)
        stage_wait(cx, dstA, pkA, sem_iA)
        _ea_unpack(pkA, wideA)
        pltpu.sync_copy(wideA, ea_acc.at[dstA], add=True)

    plsc.subcore_barrier()
    pltpu.sync_copy(ea_acc.at[pl.ds(sid * _RQ, _RQ)],
                    ea_out_hbm.at[pl.ds(cid * N + sid * _RQ, _RQ)])

    @pl.when(sid == _NS - 1)
    def _():
        pltpu.sync_copy(ea_acc.at[pl.ds(_NS * _RQ, _TAIL)],
                        ea_out_hbm.at[pl.ds(cid * N + _NS * _RQ, _TAIL)])


_ea = pl.kernel(
    _ea_body,
    out_type=jax.ShapeDtypeStruct((_NC * N, D), jnp.float32),
    mesh=_sc_mesh,
    scratch_types=[
        pltpu.VMEM((_CE,), jnp.int32),
        pltpu.VMEM((_CE,), jnp.int32),
        pltpu.VMEM((_CE // 8, D), jnp.float32),
        pltpu.VMEM((_CE // 8, D), jnp.float32),
        pltpu.VMEM((_CE, D), jnp.float32),
        pltpu.VMEM((_CE, D), jnp.float32),
        pltpu.VMEM_SHARED((N, D), jnp.float32),
        pltpu.SemaphoreType.DMA,
        pltpu.SemaphoreType.DMA,
        pltpu.SemaphoreType.DMA,
        pltpu.SemaphoreType.DMA,
    ],
)


def _mlp_body(relu_out, h_ref, p_ref, ea_ref, We_ref, W1_ref, b1_ref,
              W2_ref, b2_ref, o_ref):
    z = (h_ref[...] + p_ref[0] + p_ref[1]
         + jnp.dot(ea_ref[0] + ea_ref[1], We_ref[...],
                   preferred_element_type=jnp.float32))
    hid = jnp.maximum(
        jnp.dot(z, W1_ref[...], preferred_element_type=jnp.float32)
        + b1_ref[...], 0.0)
    z2 = (jnp.dot(hid, W2_ref[...], preferred_element_type=jnp.float32)
          + b2_ref[...])
    o_ref[...] = jnp.maximum(z2, 0.0) if relu_out else z2


def _make_mlp(relu_out):
    return pl.pallas_call(
        functools.partial(_mlp_body, relu_out),
        grid=(N // _BM,),
        in_specs=[
            pl.BlockSpec((_BM, D), lambda i: (i, 0)),
            pl.BlockSpec((_NC, _BM, D), lambda i: (0, i, 0)),
            pl.BlockSpec((_NC, _BM, DE), lambda i: (0, i, 0)),
            pl.BlockSpec((DE, D), lambda i: (0, 0)),
            pl.BlockSpec((D, 2 * D), lambda i: (0, 0)),
            pl.BlockSpec((1, 2 * D), lambda i: (0, 0)),
            pl.BlockSpec((2 * D, D), lambda i: (0, 0)),
            pl.BlockSpec((1, D), lambda i: (0, 0)),
        ],
        out_specs=pl.BlockSpec((_BM, D), lambda i: (i, 0)),
        out_shape=jax.ShapeDtypeStruct((N, D), jnp.float32),
    )


_mlp_relu = _make_mlp(True)
_mlp_last = _make_mlp(False)


def _pool_body(h_ref, b_ref, Wp_ref, bp_ref, o_ref, acc_ref, cnt_ref):
    i = pl.program_id(0)

    @pl.when(i == 0)
    def _():
        acc_ref[...] = jnp.zeros_like(acc_ref)
        cnt_ref[...] = jnp.zeros_like(cnt_ref)

    gids = lax.broadcasted_iota(jnp.int32, (G, _BM), 0)
    oh = (b_ref[0] == gids).astype(jnp.float32)
    acc_ref[...] += jnp.dot(oh, h_ref[...], preferred_element_type=jnp.float32)
    cnt_ref[...] += jnp.sum(oh, axis=1, keepdims=True)

    @pl.when(i == pl.num_programs(0) - 1)
    def _():
        emb = acc_ref[...] / jnp.maximum(cnt_ref[...], 1.0)
        o_ref[...] = (jnp.dot(emb, Wp_ref[...], preferred_element_type=jnp.float32)
                      + bp_ref[...])


_pool = pl.pallas_call(
    _pool_body,
    grid=(N // _BM,),
    in_specs=[
        pl.BlockSpec((_BM, D), lambda i: (i, 0)),
        pl.BlockSpec((1, 1, _BM), lambda i: (i, 0, 0)),
        pl.BlockSpec((D, T), lambda i: (0, 0)),
        pl.BlockSpec((1, T), lambda i: (0, 0)),
    ],
    out_specs=pl.BlockSpec((G, T), lambda i: (0, 0)),
    out_shape=jax.ShapeDtypeStruct((G, T), jnp.float32),
    scratch_shapes=[
        pltpu.VMEM((G, D), jnp.float32),
        pltpu.VMEM((G, 1), jnp.float32),
    ],
)


def kernel(x, edge_index, edge_attr, batch,
           We0, W1_0, b1_0, W2_0, b2_0,
           We1, W1_1, b1_1, W2_1, b2_1,
           Wp, bp):
    src = edge_index[0]
    dst = edge_index[1]
    zd = jnp.zeros((_RQ, D), jnp.float32)
    ea = _ea(dst, edge_attr.reshape(E // 8, 8 * DE), zd).reshape(_NC, N, D)[:, :, :DE]
    p = _seg(x, src, dst, zd).reshape(_NC, N, D)
    h1 = _mlp_relu(x, p, ea, We0, W1_0, b1_0.reshape(1, -1),
                   W2_0, b2_0.reshape(1, -1))
    p2 = _seg(h1, src, dst, zd).reshape(_NC, N, D)
    h2 = _mlp_last(h1, p2, ea, We1, W1_1, b1_1.reshape(1, -1),
                   W2_1, b2_1.reshape(1, -1))
    return _pool(h2, batch.reshape(N // _BM, 1, _BM), Wp, bp.reshape(1, -1))
